# baseline (speedup 1.0000x reference)
import jax
import jax.numpy as jnp
from jax import lax
from jax.experimental import pallas as pl
from jax.experimental.pallas import tpu as pltpu
from jax.experimental.pallas import tpu_sc as plsc

_BLOCK = 512
_N = 32768
_NW = 32
_BPW = _N // _NW

_GDN = lax.GatherDimensionNumbers(
    offset_dims=(), collapsed_slice_dims=(0,), start_index_map=(0,))


def _lane_gather(x, idx):
    return lax.gather(x, idx[:, None], _GDN, (1,),
                      mode=lax.GatherScatterMode.PROMISE_IN_BOUNDS)


def _sc_repack(rw_hbm, se_hbm, out_hbm, rw_v, se_v, r0_v, r1_v, s0_v, s1_v):
    wid = lax.axis_index("s") * 2 + lax.axis_index("c")
    base = wid * (2 * _BPW)
    pltpu.sync_copy(rw_hbm.at[pl.ds(base, 2 * _BPW)], rw_v)
    pltpu.sync_copy(se_hbm.at[pl.ds(base, 2 * _BPW)], se_v)
    iota = lax.iota(jnp.int32, 16)
    even = (iota * 2) % 16
    odd = even + 1
    lo = iota < 8

    def deint(v, j):
        a = v[pl.ds(j * 32, 16)]
        b = v[pl.ds(j * 32 + 16, 16)]
        k0 = jnp.where(lo, _lane_gather(a, even), _lane_gather(b, even))
        k1 = jnp.where(lo, _lane_gather(a, odd), _lane_gather(b, odd))
        return k0, k1

    def body(j, carry):
        sl = pl.ds(j * 16, 16)
        r0, r1 = deint(rw_v, j)
        s0, s1 = deint(se_v, j)
        r0_v[sl] = r0
        r1_v[sl] = r1
        s0_v[sl] = s0
        s1_v[sl] = s1
        return carry

    lax.fori_loop(0, _BPW // 16, body, 0)
    tsl = pl.ds(wid * _BPW, _BPW)
    pltpu.sync_copy(r0_v, out_hbm.at[0, tsl])
    pltpu.sync_copy(r1_v, out_hbm.at[1, tsl])
    pltpu.sync_copy(s0_v, out_hbm.at[2, tsl])
    pltpu.sync_copy(s1_v, out_hbm.at[3, tsl])


def _repack(rw_flat, se_flat):
    mesh = plsc.VectorSubcoreMesh(core_axis_name="c", subcore_axis_name="s")
    f = pl.kernel(
        _sc_repack,
        mesh=mesh,
        out_type=jax.ShapeDtypeStruct((4, _N), jnp.float32),
        scratch_types=[
            pltpu.VMEM((2 * _BPW,), jnp.float32),
            pltpu.VMEM((2 * _BPW,), jnp.float32),
            pltpu.VMEM((_BPW,), jnp.float32),
            pltpu.VMEM((_BPW,), jnp.float32),
            pltpu.VMEM((_BPW,), jnp.float32),
            pltpu.VMEM((_BPW,), jnp.float32),
        ],
    )
    return f(rw_flat, se_flat)


def _tc_body(ei_ref, p_ref, h_ref, o_ref):
    ei = ei_ref[0]
    a = jnp.where(p_ref[2:4, :] == ei, p_ref[0:2, :], 0.0)
    ones = jnp.ones((2, 128), jnp.float32)
    w = jax.lax.dot_general(a, ones, (((0,), (0,)), ((), ())),
                            preferred_element_type=jnp.float32)
    o_ref[...] = h_ref[...] * w[:, 0:1]


def kernel(routing_weights, selected_experts, hidden_state, expert_idx):
    n, k = routing_weights.shape
    d = hidden_state.shape[1]
    ei = jnp.asarray(expert_idx, jnp.float32).reshape((1,))
    rw_flat = routing_weights.reshape(-1)
    se_flat = selected_experts.astype(jnp.float32).reshape(-1)
    packed = _repack(rw_flat, se_flat)
    grid = (n // _BLOCK,)
    return pl.pallas_call(
        _tc_body,
        grid=grid,
        in_specs=[
            pl.BlockSpec(memory_space=pltpu.SMEM),
            pl.BlockSpec((4, _BLOCK), lambda i: (0, i)),
            pl.BlockSpec((_BLOCK, d), lambda i: (i, 0)),
        ],
        out_specs=pl.BlockSpec((_BLOCK, d), lambda i: (i, 0)),
        out_shape=jax.ShapeDtypeStruct((n, d), hidden_state.dtype),
        compiler_params=pltpu.CompilerParams(
            dimension_semantics=("arbitrary",)),
    )(ei, packed, hidden_state)


# speedup vs baseline: 1.3708x; 1.3708x over previous
import jax
import jax.numpy as jnp
from jax.experimental import pallas as pl
from jax.experimental.pallas import tpu as pltpu

_BLOCK = 512


def _body(ei_ref, rw_ref, se_ref, h_ref, o_ref):
    ei = ei_ref[0]
    a = jnp.where(se_ref[...] == ei, rw_ref[...], 0.0)
    ones = jnp.ones((2, 128), jnp.float32)
    w = jax.lax.dot_general(a, ones, (((0,), (0,)), ((), ())),
                            preferred_element_type=jnp.float32)
    o_ref[...] = h_ref[...] * w[:, 0:1]


def kernel(routing_weights, selected_experts, hidden_state, expert_idx):
    n, k = routing_weights.shape
    d = hidden_state.shape[1]
    ei = jnp.asarray(expert_idx, jnp.int32).reshape((1,))
    rw_t = routing_weights.T
    se_t = selected_experts.astype(jnp.int32).T
    grid = (n // _BLOCK,)
    return pl.pallas_call(
        _body,
        grid=grid,
        in_specs=[
            pl.BlockSpec(memory_space=pltpu.SMEM),
            pl.BlockSpec((k, _BLOCK), lambda i: (0, i)),
            pl.BlockSpec((k, _BLOCK), lambda i: (0, i)),
            pl.BlockSpec((_BLOCK, d), lambda i: (i, 0)),
        ],
        out_specs=pl.BlockSpec((_BLOCK, d), lambda i: (i, 0)),
        out_shape=jax.ShapeDtypeStruct((n, d), hidden_state.dtype),
        compiler_params=pltpu.CompilerParams(
            dimension_semantics=("arbitrary",)),
    )(ei, rw_t, se_t, hidden_state)


# speedup vs baseline: 1.3929x; 1.0162x over previous
import jax
import jax.numpy as jnp
from jax.experimental import pallas as pl
from jax.experimental.pallas import tpu as pltpu

_BLOCK = 1024


def _body(ei_ref, rw_ref, se_ref, h_ref, o_ref):
    ei = ei_ref[0]
    a = jnp.where(se_ref[...] == ei, rw_ref[...], 0.0)
    ones = jnp.ones((2, 128), jnp.float32)
    w = jax.lax.dot_general(a, ones, (((0,), (0,)), ((), ())),
                            preferred_element_type=jnp.float32)
    o_ref[...] = h_ref[...] * w[:, 0:1]


def kernel(routing_weights, selected_experts, hidden_state, expert_idx):
    n, k = routing_weights.shape
    d = hidden_state.shape[1]
    ei = jnp.asarray(expert_idx, jnp.int32).reshape((1,))
    rw_t = routing_weights.T
    se_t = selected_experts.astype(jnp.int32).T
    grid = (n // _BLOCK,)
    return pl.pallas_call(
        _body,
        grid=grid,
        in_specs=[
            pl.BlockSpec(memory_space=pltpu.SMEM),
            pl.BlockSpec((k, _BLOCK), lambda i: (0, i)),
            pl.BlockSpec((k, _BLOCK), lambda i: (0, i)),
            pl.BlockSpec((_BLOCK, d), lambda i: (i, 0)),
        ],
        out_specs=pl.BlockSpec((_BLOCK, d), lambda i: (i, 0)),
        out_shape=jax.ShapeDtypeStruct((n, d), hidden_state.dtype),
        compiler_params=pltpu.CompilerParams(
            dimension_semantics=("arbitrary",)),
    )(ei, rw_t, se_t, hidden_state)
